# Initial kernel scaffold; baseline (speedup 1.0000x reference)
import jax
import jax.numpy as jnp
from jax.experimental import pallas as pl


def kernel(x, codebooks, budget_ema):
    raise NotImplementedError("write your pallas kernel here")


# speedup vs baseline: 5.7245x; 5.7245x over previous
import jax
import jax.numpy as jnp
from jax.experimental import pallas as pl
from jax.experimental.pallas import tpu as pltpu

B, N, F = 16, 576, 768
H, M = 12, 1024
D = F // H
BN = B * N
TN = 768
GRID = BN // TN
EXPIRE_THRESHOLD = 0.05


def _vq_body(x_ref, cbT_ref, be_ref,
             out_ref, idx_ref, loss_ref, perp_ref, repl_ref,
             counts_s, loss_s):
    i = pl.program_id(0)

    @pl.when(i == 0)
    def _init():
        counts_s[...] = jnp.zeros_like(counts_s)
        loss_s[0] = jnp.float32(0.0)

    xb = x_ref[...]
    out_cols = []
    loss_acc = jnp.float32(0.0)
    for h in range(H):
        q = jax.lax.slice(xb, (0, h * D), (TN, (h + 1) * D))
        cT = cbT_ref[h]
        qc = jax.lax.dot_general(q, cT, (((1,), (0,)), ((), ())),
                                 preferred_element_type=jnp.float32)
        c2 = jnp.sum(cT * cT, axis=0, keepdims=True)
        q2 = jnp.sum(q * q, axis=1, keepdims=True)
        dist = jnp.sqrt(jnp.maximum(q2 + c2 - 2.0 * qc, 0.0))
        minv = jnp.min(dist, axis=1, keepdims=True)
        iota = jax.lax.broadcasted_iota(jnp.int32, (TN, M), 1)
        sel = jnp.where(dist == minv, iota, M)
        idxh = jnp.min(sel, axis=1).astype(jnp.int32)
        idx_ref[h, :] = idxh
        oh = (iota == idxh[:, None]).astype(jnp.float32)
        counts_s[h, :] = counts_s[h, :] + jnp.sum(oh, axis=0)
        outh = jax.lax.dot_general(oh, cT, (((1,), (1,)), ((), ())),
                                   preferred_element_type=jnp.float32)
        out_cols.append(outh)
        loss_acc = loss_acc + jnp.sum(minv * minv)

    out_ref[...] = jnp.concatenate(out_cols, axis=1)
    loss_s[0] = loss_s[0] + loss_acc

    @pl.when(i == GRID - 1)
    def _fin():
        loss_ref[0, 0] = loss_s[0] / jnp.float32(BN * F)
        mean = counts_s[...] * jnp.float32(1.0 / BN)
        ent = jnp.sum(mean * jnp.log(mean + 1e-10), axis=1)
        perp_ref[...] = jnp.broadcast_to(jnp.exp(-ent)[:, None], (H, 128))
        expired = (be_ref[...] < EXPIRE_THRESHOLD).astype(jnp.int32)
        repl_ref[...] = jnp.broadcast_to(jnp.sum(expired, axis=1)[:, None],
                                         (H, 128))


def kernel(x, codebooks, budget_ema):
    xr = x.reshape(BN, F)
    cbT = codebooks.transpose(0, 2, 1)

    out_o, idx_o, loss_o, perp_o, repl_o = pl.pallas_call(
        _vq_body,
        grid=(GRID,),
        in_specs=[
            pl.BlockSpec((TN, F), lambda i: (i, 0)),
            pl.BlockSpec((H, D, M), lambda i: (0, 0, 0)),
            pl.BlockSpec((H, M), lambda i: (0, 0)),
        ],
        out_specs=[
            pl.BlockSpec((TN, F), lambda i: (i, 0)),
            pl.BlockSpec((H, TN), lambda i: (0, i)),
            pl.BlockSpec(memory_space=pltpu.SMEM),
            pl.BlockSpec((H, 128), lambda i: (0, 0)),
            pl.BlockSpec((H, 128), lambda i: (0, 0)),
        ],
        out_shape=[
            jax.ShapeDtypeStruct((BN, F), jnp.float32),
            jax.ShapeDtypeStruct((H, BN), jnp.int32),
            jax.ShapeDtypeStruct((1, 1), jnp.float32),
            jax.ShapeDtypeStruct((H, 128), jnp.float32),
            jax.ShapeDtypeStruct((H, 128), jnp.int32),
        ],
        scratch_shapes=[
            pltpu.VMEM((H, M), jnp.float32),
            pltpu.SMEM((1,), jnp.float32),
        ],
        compiler_params=pltpu.CompilerParams(
            dimension_semantics=("arbitrary",),
        ),
    )(xr, cbT, budget_ema)

    out = out_o.reshape(B, N, F)
    codebook_indices = idx_o.reshape(H, B, N).transpose(1, 0, 2)
    loss = loss_o[0, 0]
    perp = perp_o[:, 0]
    replaced_codes = repl_o[:, 0]
    return (out, codebook_indices, loss, perp, replaced_codes, budget_ema)
